# baseline (speedup 1.0000x reference)
import functools

import jax
import jax.numpy as jnp
from jax import lax
from jax.experimental import pallas as pl
from jax.experimental.pallas import tpu as pltpu
from jax.experimental.pallas import tpu_sc as plsc

VOCAB = 1000000
DIM = 64
BATCH = 16384

NC = 2
NS = 16
NW = NC * NS
B_PER_W = BATCH // NW
CHUNK = 128
NCHUNK = B_PER_W // CHUNK
GROUPS = B_PER_W // 16


def _sc_kernel(emb_hbm, cidx_hbm, xidx_hbm, out_hbm,
               ci_v, xi_v, rows_c, rows_x, tvec, out_v, sem):
  wid = lax.axis_index("s") * NC + lax.axis_index("c")

  pltpu.sync_copy(cidx_hbm.at[wid], ci_v)
  pltpu.sync_copy(xidx_hbm.at[wid], xi_v)

  copies = []
  for j in range(NCHUNK):
    copies.append(pltpu.async_copy(
        emb_hbm.at[ci_v.at[j]], rows_c.at[pl.ds(j * CHUNK, CHUNK)], sem))
    copies.append(pltpu.async_copy(
        emb_hbm.at[xi_v.at[j]], rows_x.at[pl.ds(j * CHUNK, CHUNK)], sem))
  for cp in copies:
    cp.wait()

  lane16 = lax.iota(jnp.int32, 16) * 16

  def group_body(g, _):
    for r in range(16):
      row = g * 16 + r
      acc = None
      for d in range(DIM // 16):
        vc = rows_c[row, pl.ds(d * 16, 16)]
        vx = rows_x[row, pl.ds(d * 16, 16)]
        p = vc * vx
        acc = p if acc is None else acc + p
      plsc.store_scatter(tvec, [lane16 + r], acc)
    tot = tvec[pl.ds(0, 16)]
    for l in range(1, 16):
      tot = tot + tvec[pl.ds(l * 16, 16)]
    out_v[pl.ds(g * 16, 16)] = tot
    return _

  lax.fori_loop(0, GROUPS, group_body, None)

  pltpu.sync_copy(out_v, out_hbm.at[wid])


@jax.jit
def kernel(center_word, context_word, embedding):
  c2d = center_word.astype(jnp.int32).reshape(NW, NCHUNK, CHUNK)
  x2d = context_word.astype(jnp.int32).reshape(NW, NCHUNK, CHUNK)
  mesh = plsc.VectorSubcoreMesh(core_axis_name="c", subcore_axis_name="s")
  run = functools.partial(
      pl.kernel, mesh=mesh,
      compiler_params=pltpu.CompilerParams(
          needs_layout_passes=False, use_tc_tiling_on_sc=False),
      out_type=jax.ShapeDtypeStruct((NW, B_PER_W), jnp.float32),
      scratch_types=[
          pltpu.VMEM((NCHUNK, CHUNK), jnp.int32),
          pltpu.VMEM((NCHUNK, CHUNK), jnp.int32),
          pltpu.VMEM((B_PER_W, DIM), jnp.float32),
          pltpu.VMEM((B_PER_W, DIM), jnp.float32),
          pltpu.VMEM((256,), jnp.float32),
          pltpu.VMEM((B_PER_W,), jnp.float32),
          pltpu.SemaphoreType.DMA,
      ],
  )(_sc_kernel)
  out = run(embedding, c2d, x2d)
  return out.reshape(BATCH)


# speedup vs baseline: 3.2882x; 3.2882x over previous
import functools

import jax
import jax.numpy as jnp
from jax import lax
from jax.experimental import pallas as pl
from jax.experimental.pallas import tpu as pltpu
from jax.experimental.pallas import tpu_sc as plsc

VOCAB = 1000000
DIM = 64
BATCH = 16384

NC = 2
NS = 16
NW = NC * NS
B_PER_W = BATCH // NW

CHUNK_V = 512
NCHUNKS = 1954
LIST_CAP = 4096
HIT_CAP = 128
NIDS = 2 * BATCH
ROWS_PAD = NIDS + NW * 16


def _sweep_kernel(emb_hbm, cidx_hbm, xidx_hbm, rows_hbm,
                  ids_v, listv, listp, chbuf, hitv, hitp, rowstage, posq,
                  csem, ssem):
  wid = lax.axis_index("s") * NC + lax.axis_index("c")
  nch = 61 + (wid < 2).astype(jnp.int32)

  def fire(j):
    slot = j & 1
    c = wid + 32 * j
    base = pl.multiple_of(c * CHUNK_V, 128)
    is_last = c == (NCHUNKS - 1)

    @pl.when(jnp.logical_not(is_last))
    def _():
      pltpu.async_copy(emb_hbm.at[:, pl.ds(base, CHUNK_V)],
                       chbuf.at[slot], csem)

    @pl.when(is_last)
    def _():
      pltpu.async_copy(emb_hbm.at[:, pl.ds(base, 128)],
                       chbuf.at[slot, :, pl.ds(0, 128)], csem)

  def wait_chunk(j):
    slot = j & 1
    is_last = (wid + 32 * j) == (NCHUNKS - 1)

    @pl.when(jnp.logical_not(is_last))
    def _():
      pltpu.make_async_copy(emb_hbm.at[:, pl.ds(0, CHUNK_V)],
                            chbuf.at[slot], csem).wait()

    @pl.when(is_last)
    def _():
      pltpu.make_async_copy(emb_hbm.at[:, pl.ds(0, 128)],
                            chbuf.at[slot, :, pl.ds(0, 128)], csem).wait()

  fire(jnp.int32(0))
  fire(jnp.int32(1))

  pltpu.sync_copy(cidx_hbm, ids_v.at[pl.ds(0, BATCH)])
  pltpu.sync_copy(xidx_hbm, ids_v.at[pl.ds(BATCH, BATCH)])

  iota16 = lax.iota(jnp.int32, 16)

  def scan_body(j, cnt):
    v = ids_v[pl.ds(j * 16, 16)]
    m = ((v >> 9) & 31) == wid
    pos = iota16 + j * 16
    plsc.store_compressed(listv.at[pl.ds(cnt, 16)], v, mask=m)
    plsc.store_compressed(listp.at[pl.ds(cnt, 16)], pos, mask=m)
    return cnt + plsc.all_reduce_population_count(m)[0]

  nmatch = lax.fori_loop(0, NIDS // 16, scan_body, jnp.int32(0))
  listv[pl.ds(nmatch, 16)] = jnp.full((16,), jnp.int32(0x7fff0000))
  nvec = (nmatch + 15) >> 4

  def chunk_body(j, carry):
    c = wid + 32 * j
    wait_chunk(j)
    slot = j & 1

    def lscan(t, hcnt):
      v = listv[pl.ds(t * 16, 16)]
      p = listp[pl.ds(t * 16, 16)]
      m = (v >> 9) == c
      plsc.store_compressed(hitv.at[pl.ds(hcnt, 16)], v, mask=m)
      plsc.store_compressed(hitp.at[pl.ds(hcnt, 16)], p, mask=m)
      return jnp.minimum(hcnt + plsc.all_reduce_population_count(m)[0],
                         HIT_CAP)

    h = lax.fori_loop(0, nvec, lscan, jnp.int32(0))
    nqv = (h + 15) >> 4

    def qv_body(t, qcarry):
      qs, outst = qcarry

      @pl.when(outst == 2)
      def _():
        pltpu.make_async_copy(rowstage.at[0], rows_hbm.at[pl.ds(0, 16)],
                              ssem).wait()

      hv = hitv[pl.ds(t * 16, 16)]
      hp = hitp[pl.ds(t * 16, 16)]
      valid = (iota16 + t * 16) < h
      lc = jnp.clip(hv - c * CHUNK_V, 0, CHUNK_V - 1)
      for d in range(DIM):
        g = plsc.load_gather(chbuf.at[slot],
                             [jnp.full((16,), jnp.int32(d)), lc])
        plsc.store_scatter(rowstage.at[qs],
                           [iota16, jnp.full((16,), jnp.int32(d))], g)
      dummy = NIDS + wid * 16 + iota16
      posq[qs, pl.ds(0, 16)] = jnp.where(valid, hp, dummy)
      pltpu.async_copy(rowstage.at[qs], rows_hbm.at[posq.at[qs]], ssem)
      return 1 - qs, jnp.minimum(outst, 1) + 1

    carry = lax.fori_loop(0, nqv, qv_body, carry)

    @pl.when(j + 2 < nch)
    def _():
      fire(j + 2)

    return carry

  qs, outst = lax.fori_loop(0, nch, chunk_body,
                            (jnp.int32(0), jnp.int32(0)))

  @pl.when(outst >= 1)
  def _():
    pltpu.make_async_copy(rowstage.at[0], rows_hbm.at[pl.ds(0, 16)],
                          ssem).wait()

  @pl.when(outst == 2)
  def _():
    pltpu.make_async_copy(rowstage.at[0], rows_hbm.at[pl.ds(0, 16)],
                          ssem).wait()


def _dot_kernel(rows_hbm, out_hbm, rc_v, rx_v, tvec, out_v, sem):
  wid = lax.axis_index("s") * NC + lax.axis_index("c")
  base = wid * B_PER_W
  cp1 = pltpu.async_copy(rows_hbm.at[pl.ds(base, B_PER_W), pl.ds(0, DIM)],
                         rc_v, sem)
  cp2 = pltpu.async_copy(
      rows_hbm.at[pl.ds(BATCH + base, B_PER_W), pl.ds(0, DIM)], rx_v, sem)
  cp1.wait()
  cp2.wait()

  lane16 = lax.iota(jnp.int32, 16) * 16

  def group_body(g, _):
    for r in range(16):
      row = g * 16 + r
      acc = None
      for q in range(DIM // 16):
        vc = rc_v[row, pl.ds(q * 16, 16)]
        vx = rx_v[row, pl.ds(q * 16, 16)]
        p = vc * vx
        acc = p if acc is None else acc + p
      plsc.store_scatter(tvec, [lane16 + r], acc)
    tot = tvec[pl.ds(0, 16)]
    for l in range(1, 16):
      tot = tot + tvec[pl.ds(l * 16, 16)]
    out_v[pl.ds(g * 16, 16)] = tot
    return _

  lax.fori_loop(0, B_PER_W // 16, group_body, None)

  pltpu.sync_copy(out_v, out_hbm.at[wid])


@jax.jit
def kernel(center_word, context_word, embedding):
  emb_t = embedding.T
  ci = center_word.astype(jnp.int32)
  xi = context_word.astype(jnp.int32)
  mesh = plsc.VectorSubcoreMesh(core_axis_name="c", subcore_axis_name="s")

  sweep = functools.partial(
      pl.kernel, mesh=mesh,
      compiler_params=pltpu.CompilerParams(
          needs_layout_passes=False, disable_bounds_checks=True),
      out_type=jax.ShapeDtypeStruct((ROWS_PAD, 128), jnp.float32),
      scratch_types=[
          pltpu.VMEM((NIDS,), jnp.int32),
          pltpu.VMEM((LIST_CAP + 16,), jnp.int32),
          pltpu.VMEM((LIST_CAP + 16,), jnp.int32),
          pltpu.VMEM((2, DIM, CHUNK_V), jnp.float32),
          pltpu.VMEM((HIT_CAP + 16,), jnp.int32),
          pltpu.VMEM((HIT_CAP + 16,), jnp.int32),
          pltpu.VMEM((2, 16, 128), jnp.float32),
          pltpu.VMEM((2, 16), jnp.int32),
          pltpu.SemaphoreType.DMA,
          pltpu.SemaphoreType.DMA,
      ],
  )(_sweep_kernel)
  rows = sweep(emb_t, ci, xi)

  dot = functools.partial(
      pl.kernel, mesh=mesh,
      compiler_params=pltpu.CompilerParams(
          needs_layout_passes=False, use_tc_tiling_on_sc=False),
      out_type=jax.ShapeDtypeStruct((NW, B_PER_W), jnp.float32),
      scratch_types=[
          pltpu.VMEM((B_PER_W, DIM), jnp.float32),
          pltpu.VMEM((B_PER_W, DIM), jnp.float32),
          pltpu.VMEM((256,), jnp.float32),
          pltpu.VMEM((B_PER_W,), jnp.float32),
          pltpu.SemaphoreType.DMA,
      ],
  )(_dot_kernel)
  out = dot(rows)
  return out.reshape(BATCH)


# speedup vs baseline: 3.6571x; 1.1122x over previous
import functools

import jax
import jax.numpy as jnp
from jax import lax
from jax.experimental import pallas as pl
from jax.experimental.pallas import tpu as pltpu
from jax.experimental.pallas import tpu_sc as plsc

VOCAB = 1000000
DIM = 64
BATCH = 16384

NC = 2
NS = 16
NW = NC * NS
B_PER_W = BATCH // NW

CHUNK_V = 512
NCHUNKS = 1954
LIST_CAP = 1536
HIT_CAP = 128
NIDS = 2 * BATCH
NBUF = 3
IDS_PASS = 4096
ROWS_PAD = NIDS + NW * 16


def _sweep_kernel(emb_hbm, cidx_hbm, xidx_hbm, rows_hbm,
                  ids_v, listv, listp, chbuf, hitv, hitp, rowstage, posq,
                  csem, ssem):
  wid = lax.axis_index("s") * NC + lax.axis_index("c")
  nch = 61 + (wid < 2).astype(jnp.int32)

  def fire(j):
    slot = lax.rem(j, NBUF)
    c = wid + 32 * j
    base = pl.multiple_of(c * CHUNK_V, 128)
    is_last = c == (NCHUNKS - 1)

    @pl.when(jnp.logical_not(is_last))
    def _():
      pltpu.async_copy(emb_hbm.at[:, pl.ds(base, CHUNK_V)],
                       chbuf.at[slot], csem)

    @pl.when(is_last)
    def _():
      pltpu.async_copy(emb_hbm.at[:, pl.ds(base, 128)],
                       chbuf.at[slot, :, pl.ds(0, 128)], csem)

  def wait_chunk(j):
    slot = lax.rem(j, NBUF)
    is_last = (wid + 32 * j) == (NCHUNKS - 1)

    @pl.when(jnp.logical_not(is_last))
    def _():
      pltpu.make_async_copy(emb_hbm.at[:, pl.ds(0, CHUNK_V)],
                            chbuf.at[slot], csem).wait()

    @pl.when(is_last)
    def _():
      pltpu.make_async_copy(emb_hbm.at[:, pl.ds(0, 128)],
                            chbuf.at[slot, :, pl.ds(0, 128)], csem).wait()

  fire(jnp.int32(0))
  fire(jnp.int32(1))
  fire(jnp.int32(2))

  iota16 = lax.iota(jnp.int32, 16)

  def pass_body(p, cnt):
    off = pl.multiple_of(lax.rem(p, 4) * IDS_PASS, 128)

    @pl.when(p < 4)
    def _():
      pltpu.sync_copy(cidx_hbm.at[pl.ds(off, IDS_PASS)], ids_v)

    @pl.when(p >= 4)
    def _():
      pltpu.sync_copy(xidx_hbm.at[pl.ds(off, IDS_PASS)], ids_v)

    def scan_body(j, cnt):
      v = ids_v[pl.ds(j * 16, 16)]
      m = ((v >> 9) & 31) == wid
      pos = iota16 + p * IDS_PASS + j * 16
      plsc.store_compressed(listv.at[pl.ds(cnt, 16)], v, mask=m)
      plsc.store_compressed(listp.at[pl.ds(cnt, 16)], pos, mask=m)
      cnt = cnt + plsc.all_reduce_population_count(m)[0]
      return jnp.minimum(cnt, LIST_CAP)

    return lax.fori_loop(0, IDS_PASS // 16, scan_body, cnt)

  nmatch = lax.fori_loop(0, NIDS // IDS_PASS, pass_body, jnp.int32(0))
  listv[pl.ds(nmatch, 16)] = jnp.full((16,), jnp.int32(0x7fff0000))
  nvec = (nmatch + 15) >> 4

  def chunk_body(j, carry):
    c = wid + 32 * j
    wait_chunk(j)
    slot = lax.rem(j, NBUF)

    def lscan(t, hcnt):
      v = listv[pl.ds(t * 16, 16)]
      p = listp[pl.ds(t * 16, 16)]
      m = (v >> 9) == c
      plsc.store_compressed(hitv.at[pl.ds(hcnt, 16)], v, mask=m)
      plsc.store_compressed(hitp.at[pl.ds(hcnt, 16)], p, mask=m)
      return jnp.minimum(hcnt + plsc.all_reduce_population_count(m)[0],
                         HIT_CAP)

    h = lax.fori_loop(0, nvec, lscan, jnp.int32(0))
    nqv = (h + 15) >> 4

    def qv_body(t, qcarry):
      qs, outst = qcarry

      @pl.when(outst == 2)
      def _():
        pltpu.make_async_copy(rowstage.at[0], rows_hbm.at[pl.ds(0, 16)],
                              ssem).wait()

      hv = hitv[pl.ds(t * 16, 16)]
      hp = hitp[pl.ds(t * 16, 16)]
      valid = (iota16 + t * 16) < h
      lc = jnp.clip(hv - c * CHUNK_V, 0, CHUNK_V - 1)
      for d in range(DIM):
        g = plsc.load_gather(chbuf.at[slot],
                             [jnp.full((16,), jnp.int32(d)), lc])
        plsc.store_scatter(rowstage.at[qs],
                           [iota16, jnp.full((16,), jnp.int32(d))], g)
      dummy = NIDS + wid * 16 + iota16
      posq[qs, pl.ds(0, 16)] = jnp.where(valid, hp, dummy)
      pltpu.async_copy(rowstage.at[qs], rows_hbm.at[posq.at[qs]], ssem)
      return 1 - qs, jnp.minimum(outst, 1) + 1

    carry = lax.fori_loop(0, nqv, qv_body, carry)

    @pl.when(j + NBUF < nch)
    def _():
      fire(j + NBUF)

    return carry

  qs, outst = lax.fori_loop(0, nch, chunk_body,
                            (jnp.int32(0), jnp.int32(0)))

  @pl.when(outst >= 1)
  def _():
    pltpu.make_async_copy(rowstage.at[0], rows_hbm.at[pl.ds(0, 16)],
                          ssem).wait()

  @pl.when(outst == 2)
  def _():
    pltpu.make_async_copy(rowstage.at[0], rows_hbm.at[pl.ds(0, 16)],
                          ssem).wait()


def _dot_kernel(rows_hbm, out_hbm, rc_v, rx_v, tvec, out_v, sem):
  wid = lax.axis_index("s") * NC + lax.axis_index("c")
  base = wid * B_PER_W
  cp1 = pltpu.async_copy(rows_hbm.at[pl.ds(base, B_PER_W), pl.ds(0, DIM)],
                         rc_v, sem)
  cp2 = pltpu.async_copy(
      rows_hbm.at[pl.ds(BATCH + base, B_PER_W), pl.ds(0, DIM)], rx_v, sem)
  cp1.wait()
  cp2.wait()

  lane16 = lax.iota(jnp.int32, 16) * 16

  def group_body(g, _):
    for r in range(16):
      row = g * 16 + r
      acc = None
      for q in range(DIM // 16):
        vc = rc_v[row, pl.ds(q * 16, 16)]
        vx = rx_v[row, pl.ds(q * 16, 16)]
        p = vc * vx
        acc = p if acc is None else acc + p
      plsc.store_scatter(tvec, [lane16 + r], acc)
    tot = tvec[pl.ds(0, 16)]
    for l in range(1, 16):
      tot = tot + tvec[pl.ds(l * 16, 16)]
    out_v[pl.ds(g * 16, 16)] = tot
    return _

  lax.fori_loop(0, B_PER_W // 16, group_body, None)

  pltpu.sync_copy(out_v, out_hbm.at[wid])


@jax.jit
def kernel(center_word, context_word, embedding):
  emb_t = embedding.T
  ci = center_word.astype(jnp.int32)
  xi = context_word.astype(jnp.int32)
  mesh = plsc.VectorSubcoreMesh(core_axis_name="c", subcore_axis_name="s")

  sweep = functools.partial(
      pl.kernel, mesh=mesh,
      compiler_params=pltpu.CompilerParams(
          needs_layout_passes=False, disable_bounds_checks=True),
      out_type=jax.ShapeDtypeStruct((ROWS_PAD, 128), jnp.float32),
      scratch_types=[
          pltpu.VMEM((IDS_PASS,), jnp.int32),
          pltpu.VMEM((LIST_CAP + 16,), jnp.int32),
          pltpu.VMEM((LIST_CAP + 16,), jnp.int32),
          pltpu.VMEM((NBUF, DIM, CHUNK_V), jnp.float32),
          pltpu.VMEM((HIT_CAP + 16,), jnp.int32),
          pltpu.VMEM((HIT_CAP + 16,), jnp.int32),
          pltpu.VMEM((2, 16, 128), jnp.float32),
          pltpu.VMEM((2, 16), jnp.int32),
          pltpu.SemaphoreType.DMA,
          pltpu.SemaphoreType.DMA,
      ],
  )(_sweep_kernel)
  rows = sweep(emb_t, ci, xi)

  dot = functools.partial(
      pl.kernel, mesh=mesh,
      compiler_params=pltpu.CompilerParams(
          needs_layout_passes=False, use_tc_tiling_on_sc=False),
      out_type=jax.ShapeDtypeStruct((NW, B_PER_W), jnp.float32),
      scratch_types=[
          pltpu.VMEM((B_PER_W, DIM), jnp.float32),
          pltpu.VMEM((B_PER_W, DIM), jnp.float32),
          pltpu.VMEM((256,), jnp.float32),
          pltpu.VMEM((B_PER_W,), jnp.float32),
          pltpu.SemaphoreType.DMA,
      ],
  )(_dot_kernel)
  out = dot(rows)
  return out.reshape(BATCH)
